# Initial kernel scaffold; baseline (speedup 1.0000x reference)
import jax
import jax.numpy as jnp
from jax.experimental import pallas as pl


def kernel(input, Wp, bp, dwk, dwb, gamma, beta, Wo, bo, Wm, bm, Wout, bout):
    raise NotImplementedError("write your pallas kernel here")


# speedup vs baseline: 87.2110x; 87.2110x over previous
import functools

import jax
import jax.numpy as jnp
from jax import lax
from jax.experimental import pallas as pl
from jax.experimental.pallas import tpu as pltpu, tpu_sc as plsc

N, H, W, C = 2, 224, 224, 96
G, K, PAD = 6, 3, 1
P = K * K
GC = C // G
HB = 8
NJ = H // HB
PIX = N * H * W
B = PIX * G
NW = 32
RPW = B // NW
E = 128
NCHUNK = RPW // E
IDX_COLS = 128
IDX_ROWS = B * P // IDX_COLS


def _bf16_rne(x):
    u = lax.bitcast_convert_type(x, jnp.uint32)
    r = (u + jnp.uint32(0x7FFF) + ((u >> 16) & jnp.uint32(1))) \
        & jnp.uint32(0xFFFF0000)
    return lax.bitcast_convert_type(r, jnp.float32)


def _gelu(x):
    c = 0.7978845608028654
    return 0.5 * x * (1 + jnp.tanh(c * (x + 0.044715 * x ** 3)))


def _head_body(in_hbm, Wp, bp, dwk, dwb, gamma, beta, Woy, boy, Wox, box,
               Wm, bm, xo, idxo, masko, inp_v, sem):
    n = pl.program_id(0)
    j = pl.program_id(1)
    r0 = j * HB

    @pl.when(j == 0)
    def _():
        cp = pltpu.make_async_copy(in_hbm.at[n, pl.ds(0, HB + 1)],
                                   inp_v.at[pl.ds(1, HB + 1)], sem)
        cp.start()
        cp.wait()
        inp_v[0, :, :] = jnp.zeros((W, C), jnp.float32)

    @pl.when(j == NJ - 1)
    def _():
        cp = pltpu.make_async_copy(in_hbm.at[n, pl.ds(H - HB - 1, HB + 1)],
                                   inp_v.at[pl.ds(0, HB + 1)], sem)
        cp.start()
        cp.wait()
        inp_v[HB + 1, :, :] = jnp.zeros((W, C), jnp.float32)

    @pl.when((j > 0) & (j < NJ - 1))
    def _():
        cp = pltpu.make_async_copy(in_hbm.at[n, pl.ds(r0 - 1, HB + 2)],
                                   inp_v.at[:], sem)
        cp.start()
        cp.wait()

    xin = inp_v[...]

    xrp = _bf16_rne(xin)
    acc = jnp.zeros((HB, W, C), jnp.float32)
    zcol = jnp.zeros((HB, 1, C), jnp.float32)
    for dy in range(3):
        rows = xrp[dy:dy + HB]
        for dx in range(3):
            if dx == 0:
                s = jnp.concatenate([zcol, rows[:, :W - 1]], axis=1)
            elif dx == 1:
                s = rows
            else:
                s = jnp.concatenate([rows[:, 1:], zcol], axis=1)
            acc = acc + s * dwk[dy * 3 + dx]
    acc = acc + dwb[0]

    mu = jnp.mean(acc, -1, keepdims=True)
    var = jnp.mean((acc - mu) ** 2, -1, keepdims=True)
    x1 = (acc - mu) / jnp.sqrt(var + 1e-6) * gamma[0] + beta[0]
    x1 = _gelu(x1)
    x1f = x1.reshape(HB * W, C)

    center = xin[1:1 + HB].reshape(HB * W, C)
    xo[...] = jnp.dot(center, Wp[...],
                      preferred_element_type=jnp.float32, precision=lax.Precision.DEFAULT) + bp[0]

    offy = jnp.dot(x1f, Woy[...], preferred_element_type=jnp.float32, precision=lax.Precision.DEFAULT) + boy[0]
    offx = jnp.dot(x1f, Wox[...], preferred_element_type=jnp.float32, precision=lax.Precision.DEFAULT) + box[0]

    Z = jnp.dot(x1f, Wm[...], preferred_element_type=jnp.float32, precision=lax.Precision.DEFAULT) + bm[0]
    Mx = jnp.max(Z, -1, keepdims=True)
    Ez = jnp.exp(Z - Mx)
    ii = lax.broadcasted_iota(jnp.int32, (G * P, G * P), 0) // P
    jj = lax.broadcasted_iota(jnp.int32, (G * P, G * P), 1) // P
    bd = (ii == jj).astype(jnp.float32)
    den = jnp.dot(Ez, bd, preferred_element_type=jnp.float32, precision=lax.Precision.HIGHEST)
    msk = Ez / den

    ri = lax.broadcasted_iota(jnp.int32, (HB * W, 1), 0)
    hh = r0 + ri // W
    ww = ri % W
    refy = ww.astype(jnp.float32) + 1.5
    refx = hh.astype(jnp.float32) + 1.5
    kk = lax.broadcasted_iota(jnp.int32, (HB * W, G * P), 1)
    pp = kk % P
    gg = kk // P
    gy = (pp // 3 - 1).astype(jnp.float32)
    gx = (pp % 3 - 1).astype(jnp.float32)
    iy = jnp.clip((refy + gy + offy).astype(jnp.int32), 0, H + 1)
    ix = jnp.clip((refx + gx + offx).astype(jnp.int32), 0, W + 1)
    valid = (iy >= 1) & (iy <= H) & (ix >= 1) & (ix <= W)
    iyu = jnp.clip(iy - 1, 0, H - 1)
    ixu = jnp.clip(ix - 1, 0, W - 1)
    idxo[...] = ((n * H + iyu) * W + ixu) * G + gg
    masko[...] = msk * valid.astype(jnp.float32)


def _head(input, Wp, bp, dwk9, dwb, gamma, beta, Woy, boy, Wox, box, Wm, bm):
    blk = lambda: pl.BlockSpec((HB * W, G * P), lambda n, j: (n * NJ + j, 0))
    full = lambda a: pl.BlockSpec(a.shape, lambda n, j: (0,) * a.ndim)
    weights = (Wp, bp, dwk9, dwb, gamma, beta, Woy, boy, Wox, box, Wm, bm)
    return pl.pallas_call(
        _head_body,
        grid=(N, NJ),
        in_specs=[pl.BlockSpec(memory_space=pltpu.MemorySpace.HBM)]
        + [full(w) for w in weights],
        out_specs=[
            pl.BlockSpec((HB * W, C), lambda n, j: (n * NJ + j, 0)),
            blk(),
            blk(),
        ],
        out_shape=[
            jax.ShapeDtypeStruct((PIX, C), jnp.float32),
            jax.ShapeDtypeStruct((PIX, G * P), jnp.int32),
            jax.ShapeDtypeStruct((PIX, G * P), jnp.float32),
        ],
        scratch_shapes=[
            pltpu.VMEM((HB + 2, W, C), jnp.float32),
            pltpu.SemaphoreType.DMA,
        ],
    )(input, *weights)


def _sc_body(table, idxf, maskf, out, idx_v, rows_v, mask_v, out_v, gsem):
    nc = 2
    wid = lax.axis_index("s") * nc + lax.axis_index("c")

    def chunk(c, _):
        base = wid * RPW + c * E
        pltpu.sync_copy(idxf.at[pl.ds(base * P, E * P)], idx_v)
        pltpu.sync_copy(maskf.at[pl.ds(base * P, E * P)],
                        mask_v.at[pl.ds(0, E * P)])
        cps = [pltpu.async_copy(table.at[idx_v.at[pl.ds(q * IDX_COLS,
                                                        IDX_COLS)]],
                                rows_v.at[pl.ds(q * IDX_COLS, IDX_COLS)], gsem)
               for q in range(P)]
        for cp in cps:
            cp.wait()

        lanes = lax.iota(jnp.int32, GC)
        dnums = lax.GatherDimensionNumbers(
            offset_dims=(), collapsed_slice_dims=(0,), start_index_map=(0,))

        def lane_bcast(v, p):
            idx = jnp.full((GC, 1), p, jnp.int32)
            return lax.gather(v, idx, dnums, (1,),
                              mode=lax.GatherScatterMode.PROMISE_IN_BOUNDS)

        def erow(e, _):
            mvec = plsc.load_gather(mask_v, [e * P + lanes])
            acc = jnp.zeros((GC,), jnp.float32)
            for p in range(P):
                acc = acc + rows_v[e * P + p] * lane_bcast(mvec, p)
            out_v[e] = acc
            return 0

        lax.fori_loop(0, E, erow, 0)
        pltpu.sync_copy(out_v, out.at[pl.ds(base, E)])
        return 0

    lax.fori_loop(0, NCHUNK, chunk, 0)


def _sc_gather(table, idxf, maskf):
    mesh = plsc.VectorSubcoreMesh(core_axis_name="c", subcore_axis_name="s")
    return pl.kernel(
        _sc_body,
        out_type=jax.ShapeDtypeStruct((B, GC), jnp.float32),
        mesh=mesh,
        compiler_params=pltpu.CompilerParams(needs_layout_passes=False,
                                             use_tc_tiling_on_sc=False),
        scratch_types=[
            pltpu.VMEM((E * P,), jnp.int32),
            pltpu.VMEM((E * P, GC), jnp.float32),
            pltpu.VMEM((E * P + GC,), jnp.float32),
            pltpu.VMEM((E, GC), jnp.float32),
            pltpu.SemaphoreType.DMA,
        ],
    )(table, idxf, maskf)


def _tail_body(y, Wout, bout, out):
    out[...] = jnp.dot(y[...], Wout[...],
                       preferred_element_type=jnp.float32, precision=lax.Precision.DEFAULT) + bout[0]


def _tail(y, Wout, bout):
    RB = 2048
    return pl.pallas_call(
        _tail_body,
        grid=(PIX // RB,),
        in_specs=[
            pl.BlockSpec((RB, C), lambda i: (i, 0)),
            pl.BlockSpec((C, C), lambda i: (0, 0)),
            pl.BlockSpec((1, C), lambda i: (0, 0)),
        ],
        out_specs=pl.BlockSpec((RB, C), lambda i: (i, 0)),
        out_shape=jax.ShapeDtypeStruct((PIX, C), jnp.float32),
    )(y, Wout, bout)


def kernel(input, Wp, bp, dwk, dwb, gamma, beta, Wo, bo, Wm, bm, Wout, bout):
    Woy, Wox = Wo[:, 0::2], Wo[:, 1::2]
    boy, box = bo[0::2], bo[1::2]
    r1 = lambda v: v.reshape(1, -1)
    xo, idxo, masko = _head(input, Wp, r1(bp), dwk.reshape(P, C), r1(dwb),
                            r1(gamma), r1(beta), Woy, r1(boy), Wox, r1(box),
                            Wm, r1(bm))
    out_core = _sc_gather(xo.reshape(B, GC),
                          idxo.reshape(B * P),
                          masko.reshape(B * P))
    y = _tail(out_core.reshape(PIX, C), Wout, r1(bout))
    return y.reshape(N, H, W, C)


# speedup vs baseline: 115.4894x; 1.3243x over previous
import functools

import jax
import jax.numpy as jnp
from jax import lax
from jax.experimental import pallas as pl
from jax.experimental.pallas import tpu as pltpu, tpu_sc as plsc

N, H, W, C = 2, 224, 224, 96
G, K, PAD = 6, 3, 1
P = K * K
GC = C // G
HB = 8
NJ = H // HB
PIX = N * H * W
B = PIX * G
NW = 32
RPW = B // NW
E = 128
NCHUNK = RPW // E
IDX_COLS = 128
IDX_ROWS = B * P // IDX_COLS


def _bf16_rne(x):
    u = lax.bitcast_convert_type(x, jnp.uint32)
    r = (u + jnp.uint32(0x7FFF) + ((u >> 16) & jnp.uint32(1))) \
        & jnp.uint32(0xFFFF0000)
    return lax.bitcast_convert_type(r, jnp.float32)


def _gelu(x):
    c = 0.7978845608028654
    return 0.5 * x * (1 + jnp.tanh(c * (x + 0.044715 * x ** 3)))


def _head_body(in_hbm, Wp, bp, dwk, dwb, gamma, beta, Woy, boy, Wox, box,
               Wm, bm, xo, idxo, masko, inp_v, sem):
    n = pl.program_id(0)
    j = pl.program_id(1)
    r0 = j * HB

    @pl.when(j == 0)
    def _():
        cp = pltpu.make_async_copy(in_hbm.at[n, pl.ds(0, HB + 1)],
                                   inp_v.at[pl.ds(1, HB + 1)], sem)
        cp.start()
        cp.wait()
        inp_v[0, :, :] = jnp.zeros((W, C), jnp.float32)

    @pl.when(j == NJ - 1)
    def _():
        cp = pltpu.make_async_copy(in_hbm.at[n, pl.ds(H - HB - 1, HB + 1)],
                                   inp_v.at[pl.ds(0, HB + 1)], sem)
        cp.start()
        cp.wait()
        inp_v[HB + 1, :, :] = jnp.zeros((W, C), jnp.float32)

    @pl.when((j > 0) & (j < NJ - 1))
    def _():
        cp = pltpu.make_async_copy(in_hbm.at[n, pl.ds(r0 - 1, HB + 2)],
                                   inp_v.at[:], sem)
        cp.start()
        cp.wait()

    xin = inp_v[...]

    xrp = _bf16_rne(xin)
    acc = jnp.zeros((HB, W, C), jnp.float32)
    zcol = jnp.zeros((HB, 1, C), jnp.float32)
    for dy in range(3):
        rows = xrp[dy:dy + HB]
        for dx in range(3):
            if dx == 0:
                s = jnp.concatenate([zcol, rows[:, :W - 1]], axis=1)
            elif dx == 1:
                s = rows
            else:
                s = jnp.concatenate([rows[:, 1:], zcol], axis=1)
            acc = acc + s * dwk[dy * 3 + dx]
    acc = acc + dwb[0]

    mu = jnp.mean(acc, -1, keepdims=True)
    var = jnp.mean((acc - mu) ** 2, -1, keepdims=True)
    x1 = (acc - mu) / jnp.sqrt(var + 1e-6) * gamma[0] + beta[0]
    x1 = _gelu(x1)
    x1f = x1.reshape(HB * W, C)

    center = xin[1:1 + HB].reshape(HB * W, C)
    xo[...] = jnp.dot(center, Wp[...],
                      preferred_element_type=jnp.float32, precision=lax.Precision.DEFAULT) + bp[0]

    offy = jnp.dot(x1f, Woy[...], preferred_element_type=jnp.float32, precision=lax.Precision.DEFAULT) + boy[0]
    offx = jnp.dot(x1f, Wox[...], preferred_element_type=jnp.float32, precision=lax.Precision.DEFAULT) + box[0]

    Z = jnp.dot(x1f, Wm[...], preferred_element_type=jnp.float32, precision=lax.Precision.DEFAULT) + bm[0]
    Mx = jnp.max(Z, -1, keepdims=True)
    Ez = jnp.exp(Z - Mx)
    ii = lax.broadcasted_iota(jnp.int32, (G * P, G * P), 0) // P
    jj = lax.broadcasted_iota(jnp.int32, (G * P, G * P), 1) // P
    bd = (ii == jj).astype(jnp.float32)
    den = jnp.dot(Ez, bd, preferred_element_type=jnp.float32, precision=lax.Precision.HIGHEST)
    msk = Ez / den

    ri = lax.broadcasted_iota(jnp.int32, (HB * W, 1), 0)
    hh = r0 + ri // W
    ww = ri % W
    refy = ww.astype(jnp.float32) + 1.5
    refx = hh.astype(jnp.float32) + 1.5
    kk = lax.broadcasted_iota(jnp.int32, (HB * W, G * P), 1)
    pp = kk % P
    gg = kk // P
    gy = (pp // 3 - 1).astype(jnp.float32)
    gx = (pp % 3 - 1).astype(jnp.float32)
    iy = jnp.clip((refy + gy + offy).astype(jnp.int32), 0, H + 1)
    ix = jnp.clip((refx + gx + offx).astype(jnp.int32), 0, W + 1)
    valid = (iy >= 1) & (iy <= H) & (ix >= 1) & (ix <= W)
    iyu = jnp.clip(iy - 1, 0, H - 1)
    ixu = jnp.clip(ix - 1, 0, W - 1)
    idxo[...] = ((n * H + iyu) * W + ixu) * G + gg
    masko[...] = msk * valid.astype(jnp.float32)


def _head(input, Wp, bp, dwk9, dwb, gamma, beta, Woy, boy, Wox, box, Wm, bm):
    blk = lambda: pl.BlockSpec((HB * W, G * P), lambda n, j: (n * NJ + j, 0))
    full = lambda a: pl.BlockSpec(a.shape, lambda n, j: (0,) * a.ndim)
    weights = (Wp, bp, dwk9, dwb, gamma, beta, Woy, boy, Wox, box, Wm, bm)
    return pl.pallas_call(
        _head_body,
        grid=(N, NJ),
        in_specs=[pl.BlockSpec(memory_space=pltpu.MemorySpace.HBM)]
        + [full(w) for w in weights],
        out_specs=[
            pl.BlockSpec((HB * W, C), lambda n, j: (n * NJ + j, 0)),
            blk(),
            blk(),
        ],
        out_shape=[
            jax.ShapeDtypeStruct((PIX, C), jnp.float32),
            jax.ShapeDtypeStruct((PIX, G * P), jnp.int32),
            jax.ShapeDtypeStruct((PIX, G * P), jnp.float32),
        ],
        scratch_shapes=[
            pltpu.VMEM((HB + 2, W, C), jnp.float32),
            pltpu.SemaphoreType.DMA,
        ],
    )(input, *weights)


def _sc_body(table, idxf, maskf, out, idx_v, rows_v, mask_v, out_v,
             isem, gsem, osem):
    nc = 2
    wid = lax.axis_index("s") * nc + lax.axis_index("c")

    def stage_in(c, b):
        base = wid * RPW + c * E
        pltpu.async_copy(idxf.at[pl.ds(base * P, E * P)], idx_v.at[b], isem)
        pltpu.async_copy(maskf.at[pl.ds(base * P, E * P)],
                         mask_v.at[b, pl.ds(0, E * P)], isem)

    def wait_in(c, b):
        base = wid * RPW + c * E
        pltpu.make_async_copy(idxf.at[pl.ds(base * P, E * P)],
                              idx_v.at[b], isem).wait()
        pltpu.make_async_copy(maskf.at[pl.ds(base * P, E * P)],
                              mask_v.at[b, pl.ds(0, E * P)], isem).wait()

    def fire_gathers(b):
        for q in range(P):
            pltpu.async_copy(
                table.at[idx_v.at[b, pl.ds(q * IDX_COLS, IDX_COLS)]],
                rows_v.at[b, pl.ds(q * IDX_COLS, IDX_COLS)], gsem)

    def drain_gathers(b):
        for q in range(P):
            pltpu.make_async_copy(
                table.at[idx_v.at[b, pl.ds(q * IDX_COLS, IDX_COLS)]],
                rows_v.at[b, pl.ds(q * IDX_COLS, IDX_COLS)], gsem).wait()

    lanes = lax.iota(jnp.int32, GC)
    dnums = lax.GatherDimensionNumbers(
        offset_dims=(), collapsed_slice_dims=(0,), start_index_map=(0,))

    def lane_bcast(v, p):
        idx = jnp.full((GC, 1), p, jnp.int32)
        return lax.gather(v, idx, dnums, (1,),
                          mode=lax.GatherScatterMode.PROMISE_IN_BOUNDS)

    stage_in(0, 0)
    wait_in(0, 0)
    fire_gathers(0)
    stage_in(1, 1)

    def chunk(c, _):
        b = lax.rem(c, 2)
        nb = 1 - b
        base = wid * RPW + c * E

        drain_gathers(b)

        @pl.when(c + 1 < NCHUNK)
        def _():
            wait_in(c + 1, nb)
            fire_gathers(nb)

        @pl.when(c >= 2)
        def _():
            pltpu.make_async_copy(
                out_v.at[b], out.at[pl.ds(base - 2 * E, E)], osem).wait()

        def erow(e, _):
            mvec = plsc.load_gather(mask_v.at[b], [e * P + lanes])
            acc = jnp.zeros((GC,), jnp.float32)
            for p in range(P):
                acc = acc + rows_v[b, e * P + p] * lane_bcast(mvec, p)
            out_v[b, e] = acc
            return 0

        lax.fori_loop(0, E, erow, 0, unroll=2)
        pltpu.async_copy(out_v.at[b], out.at[pl.ds(base, E)], osem)

        @pl.when(c + 2 < NCHUNK)
        def _():
            stage_in(c + 2, b)

        return 0

    lax.fori_loop(0, NCHUNK, chunk, 0)
    for c in (NCHUNK - 2, NCHUNK - 1):
        b = c % 2
        base = wid * RPW + c * E
        pltpu.make_async_copy(out_v.at[b], out.at[pl.ds(base, E)],
                              osem).wait()


def _sc_gather(table, idxf, maskf):
    mesh = plsc.VectorSubcoreMesh(core_axis_name="c", subcore_axis_name="s")
    return pl.kernel(
        _sc_body,
        out_type=jax.ShapeDtypeStruct((B, GC), jnp.float32),
        mesh=mesh,
        compiler_params=pltpu.CompilerParams(needs_layout_passes=False,
                                             use_tc_tiling_on_sc=False),
        scratch_types=[
            pltpu.VMEM((2, E * P), jnp.int32),
            pltpu.VMEM((2, E * P, GC), jnp.float32),
            pltpu.VMEM((2, E * P + GC), jnp.float32),
            pltpu.VMEM((2, E, GC), jnp.float32),
            pltpu.SemaphoreType.DMA,
            pltpu.SemaphoreType.DMA,
            pltpu.SemaphoreType.DMA,
        ],
    )(table, idxf, maskf)


def _tail_body(y, Wout, bout, out):
    out[...] = jnp.dot(y[...], Wout[...],
                       preferred_element_type=jnp.float32, precision=lax.Precision.DEFAULT) + bout[0]


def _tail(y, Wout, bout):
    RB = 2048
    return pl.pallas_call(
        _tail_body,
        grid=(PIX // RB,),
        in_specs=[
            pl.BlockSpec((RB, C), lambda i: (i, 0)),
            pl.BlockSpec((C, C), lambda i: (0, 0)),
            pl.BlockSpec((1, C), lambda i: (0, 0)),
        ],
        out_specs=pl.BlockSpec((RB, C), lambda i: (i, 0)),
        out_shape=jax.ShapeDtypeStruct((PIX, C), jnp.float32),
    )(y, Wout, bout)


def kernel(input, Wp, bp, dwk, dwb, gamma, beta, Wo, bo, Wm, bm, Wout, bout):
    Woy, Wox = Wo[:, 0::2], Wo[:, 1::2]
    boy, box = bo[0::2], bo[1::2]
    r1 = lambda v: v.reshape(1, -1)
    xo, idxo, masko = _head(input, Wp, r1(bp), dwk.reshape(P, C), r1(dwb),
                            r1(gamma), r1(beta), Woy, r1(boy), Wox, r1(box),
                            Wm, r1(bm))
    out_core = _sc_gather(xo.reshape(B, GC),
                          idxo.reshape(B * P),
                          masko.reshape(B * P))
    y = _tail(out_core.reshape(PIX, C), Wout, r1(bout))
    return y.reshape(N, H, W, C)


# speedup vs baseline: 119.6674x; 1.0362x over previous
import functools

import jax
import jax.numpy as jnp
from jax import lax
from jax.experimental import pallas as pl
from jax.experimental.pallas import tpu as pltpu, tpu_sc as plsc

N, H, W, C = 2, 224, 224, 96
G, K, PAD = 6, 3, 1
P = K * K
GC = C // G
HB = 16
NJ = H // HB
PIX = N * H * W
B = PIX * G
NW = 32
RPW = B // NW
E = 128
NCHUNK = RPW // E
IDX_COLS = 128
IDX_ROWS = B * P // IDX_COLS


def _bf16_rne(x):
    u = lax.bitcast_convert_type(x, jnp.uint32)
    r = (u + jnp.uint32(0x7FFF) + ((u >> 16) & jnp.uint32(1))) \
        & jnp.uint32(0xFFFF0000)
    return lax.bitcast_convert_type(r, jnp.float32)


def _gelu(x):
    c = 0.7978845608028654
    return 0.5 * x * (1 + jnp.tanh(c * (x + 0.044715 * x ** 3)))


def _head_body(in_hbm, Wp, bp, dwk, dwb, gamma, beta, Woy, boy, Wox, box,
               Wm, bm, xo, idxo, masko, inp_v, sem):
    n = pl.program_id(0)
    j = pl.program_id(1)
    r0 = j * HB

    @pl.when(j == 0)
    def _():
        cp = pltpu.make_async_copy(in_hbm.at[n, pl.ds(0, HB + 1)],
                                   inp_v.at[pl.ds(1, HB + 1)], sem)
        cp.start()
        cp.wait()
        inp_v[0, :, :] = jnp.zeros((W, C), jnp.float32)

    @pl.when(j == NJ - 1)
    def _():
        cp = pltpu.make_async_copy(in_hbm.at[n, pl.ds(H - HB - 1, HB + 1)],
                                   inp_v.at[pl.ds(0, HB + 1)], sem)
        cp.start()
        cp.wait()
        inp_v[HB + 1, :, :] = jnp.zeros((W, C), jnp.float32)

    @pl.when((j > 0) & (j < NJ - 1))
    def _():
        cp = pltpu.make_async_copy(in_hbm.at[n, pl.ds(r0 - 1, HB + 2)],
                                   inp_v.at[:], sem)
        cp.start()
        cp.wait()

    xin = inp_v[...]

    xrp = _bf16_rne(xin)
    acc = jnp.zeros((HB, W, C), jnp.float32)
    zcol = jnp.zeros((HB, 1, C), jnp.float32)
    for dy in range(3):
        rows = xrp[dy:dy + HB]
        for dx in range(3):
            if dx == 0:
                s = jnp.concatenate([zcol, rows[:, :W - 1]], axis=1)
            elif dx == 1:
                s = rows
            else:
                s = jnp.concatenate([rows[:, 1:], zcol], axis=1)
            acc = acc + s * dwk[dy * 3 + dx]
    acc = acc + dwb[0]

    mu = jnp.mean(acc, -1, keepdims=True)
    var = jnp.mean((acc - mu) ** 2, -1, keepdims=True)
    x1 = (acc - mu) / jnp.sqrt(var + 1e-6) * gamma[0] + beta[0]
    x1 = _gelu(x1)
    x1f = x1.reshape(HB * W, C)

    center = xin[1:1 + HB].reshape(HB * W, C)
    xo[...] = jnp.dot(center, Wp[...],
                      preferred_element_type=jnp.float32, precision=lax.Precision.DEFAULT) + bp[0]

    offy = jnp.dot(x1f, Woy[...], preferred_element_type=jnp.float32, precision=lax.Precision.DEFAULT) + boy[0]
    offx = jnp.dot(x1f, Wox[...], preferred_element_type=jnp.float32, precision=lax.Precision.DEFAULT) + box[0]

    Z = jnp.dot(x1f, Wm[...], preferred_element_type=jnp.float32, precision=lax.Precision.DEFAULT) + bm[0]
    Mx = jnp.max(Z, -1, keepdims=True)
    Ez = jnp.exp(Z - Mx)
    ii = lax.broadcasted_iota(jnp.int32, (G * P, G * P), 0) // P
    jj = lax.broadcasted_iota(jnp.int32, (G * P, G * P), 1) // P
    bd = (ii == jj).astype(jnp.float32)
    den = jnp.dot(Ez, bd, preferred_element_type=jnp.float32, precision=lax.Precision.HIGHEST)
    msk = Ez / den

    ri = lax.broadcasted_iota(jnp.int32, (HB * W, 1), 0)
    hh = r0 + ri // W
    ww = ri % W
    refy = ww.astype(jnp.float32) + 1.5
    refx = hh.astype(jnp.float32) + 1.5
    kk = lax.broadcasted_iota(jnp.int32, (HB * W, G * P), 1)
    pp = kk % P
    gg = kk // P
    gy = (pp // 3 - 1).astype(jnp.float32)
    gx = (pp % 3 - 1).astype(jnp.float32)
    iy = jnp.clip((refy + gy + offy).astype(jnp.int32), 0, H + 1)
    ix = jnp.clip((refx + gx + offx).astype(jnp.int32), 0, W + 1)
    valid = (iy >= 1) & (iy <= H) & (ix >= 1) & (ix <= W)
    iyu = jnp.clip(iy - 1, 0, H - 1)
    ixu = jnp.clip(ix - 1, 0, W - 1)
    idxo[...] = ((n * H + iyu) * W + ixu) * G + gg
    masko[...] = msk * valid.astype(jnp.float32)


def _head(input, Wp, bp, dwk9, dwb, gamma, beta, Woy, boy, Wox, box, Wm, bm):
    blk = lambda: pl.BlockSpec((HB * W, G * P), lambda n, j: (n * NJ + j, 0))
    full = lambda a: pl.BlockSpec(a.shape, lambda n, j: (0,) * a.ndim)
    weights = (Wp, bp, dwk9, dwb, gamma, beta, Woy, boy, Wox, box, Wm, bm)
    return pl.pallas_call(
        _head_body,
        grid=(N, NJ),
        in_specs=[pl.BlockSpec(memory_space=pltpu.MemorySpace.HBM)]
        + [full(w) for w in weights],
        out_specs=[
            pl.BlockSpec((HB * W, C), lambda n, j: (n * NJ + j, 0)),
            blk(),
            blk(),
        ],
        out_shape=[
            jax.ShapeDtypeStruct((PIX, C), jnp.float32),
            jax.ShapeDtypeStruct((PIX, G * P), jnp.int32),
            jax.ShapeDtypeStruct((PIX, G * P), jnp.float32),
        ],
        scratch_shapes=[
            pltpu.VMEM((HB + 2, W, C), jnp.float32),
            pltpu.SemaphoreType.DMA,
        ],
    )(input, *weights)


def _sc_body(table, idxf, maskf, out, idx_v, rows_v, mask_v, out_v,
             isem, gsem, osem):
    nc = 2
    wid = lax.axis_index("s") * nc + lax.axis_index("c")

    def stage_in(c, b):
        base = wid * RPW + c * E
        pltpu.async_copy(idxf.at[pl.ds(base * P, E * P)], idx_v.at[b], isem)
        pltpu.async_copy(maskf.at[pl.ds(base * P, E * P)],
                         mask_v.at[b, pl.ds(0, E * P)], isem)

    def wait_in(c, b):
        base = wid * RPW + c * E
        pltpu.make_async_copy(idxf.at[pl.ds(base * P, E * P)],
                              idx_v.at[b], isem).wait()
        pltpu.make_async_copy(maskf.at[pl.ds(base * P, E * P)],
                              mask_v.at[b, pl.ds(0, E * P)], isem).wait()

    def fire_gathers(b):
        for q in range(P):
            pltpu.async_copy(
                table.at[idx_v.at[b, pl.ds(q * IDX_COLS, IDX_COLS)]],
                rows_v.at[b, pl.ds(q * IDX_COLS, IDX_COLS)], gsem)

    def drain_gathers(b):
        for q in range(P):
            pltpu.make_async_copy(
                table.at[idx_v.at[b, pl.ds(q * IDX_COLS, IDX_COLS)]],
                rows_v.at[b, pl.ds(q * IDX_COLS, IDX_COLS)], gsem).wait()

    lanes = lax.iota(jnp.int32, GC)
    dnums = lax.GatherDimensionNumbers(
        offset_dims=(), collapsed_slice_dims=(0,), start_index_map=(0,))

    def lane_bcast(v, p):
        idx = jnp.full((GC, 1), p, jnp.int32)
        return lax.gather(v, idx, dnums, (1,),
                          mode=lax.GatherScatterMode.PROMISE_IN_BOUNDS)

    stage_in(0, 0)
    wait_in(0, 0)
    fire_gathers(0)
    stage_in(1, 1)

    def chunk(c, _):
        b = lax.rem(c, 2)
        nb = 1 - b
        base = wid * RPW + c * E

        drain_gathers(b)

        @pl.when(c + 1 < NCHUNK)
        def _():
            wait_in(c + 1, nb)
            fire_gathers(nb)

        @pl.when(c >= 2)
        def _():
            pltpu.make_async_copy(
                out_v.at[b], out.at[pl.ds(base - 2 * E, E)], osem).wait()

        def erow(e, _):
            mvec = plsc.load_gather(mask_v.at[b], [e * P + lanes])
            acc = jnp.zeros((GC,), jnp.float32)
            for p in range(P):
                acc = acc + rows_v[b, e * P + p] * lane_bcast(mvec, p)
            out_v[b, e] = acc
            return 0

        lax.fori_loop(0, E, erow, 0, unroll=2)
        pltpu.async_copy(out_v.at[b], out.at[pl.ds(base, E)], osem)

        @pl.when(c + 2 < NCHUNK)
        def _():
            stage_in(c + 2, b)

        return 0

    lax.fori_loop(0, NCHUNK, chunk, 0)
    for c in (NCHUNK - 2, NCHUNK - 1):
        b = c % 2
        base = wid * RPW + c * E
        pltpu.make_async_copy(out_v.at[b], out.at[pl.ds(base, E)],
                              osem).wait()


def _sc_gather(table, idxf, maskf):
    mesh = plsc.VectorSubcoreMesh(core_axis_name="c", subcore_axis_name="s")
    return pl.kernel(
        _sc_body,
        out_type=jax.ShapeDtypeStruct((B, GC), jnp.float32),
        mesh=mesh,
        compiler_params=pltpu.CompilerParams(needs_layout_passes=False,
                                             use_tc_tiling_on_sc=False),
        scratch_types=[
            pltpu.VMEM((2, E * P), jnp.int32),
            pltpu.VMEM((2, E * P, GC), jnp.float32),
            pltpu.VMEM((2, E * P + GC), jnp.float32),
            pltpu.VMEM((2, E, GC), jnp.float32),
            pltpu.SemaphoreType.DMA,
            pltpu.SemaphoreType.DMA,
            pltpu.SemaphoreType.DMA,
        ],
    )(table, idxf, maskf)


def _tail_body(y, Wout, bout, out):
    out[...] = jnp.dot(y[...], Wout[...],
                       preferred_element_type=jnp.float32, precision=lax.Precision.DEFAULT) + bout[0]


def _tail(y, Wout, bout):
    RB = 2048
    return pl.pallas_call(
        _tail_body,
        grid=(PIX // RB,),
        in_specs=[
            pl.BlockSpec((RB, C), lambda i: (i, 0)),
            pl.BlockSpec((C, C), lambda i: (0, 0)),
            pl.BlockSpec((1, C), lambda i: (0, 0)),
        ],
        out_specs=pl.BlockSpec((RB, C), lambda i: (i, 0)),
        out_shape=jax.ShapeDtypeStruct((PIX, C), jnp.float32),
    )(y, Wout, bout)


def kernel(input, Wp, bp, dwk, dwb, gamma, beta, Wo, bo, Wm, bm, Wout, bout):
    Woy, Wox = Wo[:, 0::2], Wo[:, 1::2]
    boy, box = bo[0::2], bo[1::2]
    r1 = lambda v: v.reshape(1, -1)
    xo, idxo, masko = _head(input, Wp, r1(bp), dwk.reshape(P, C), r1(dwb),
                            r1(gamma), r1(beta), Woy, r1(boy), Wox, r1(box),
                            Wm, r1(bm))
    out_core = _sc_gather(xo.reshape(B, GC),
                          idxo.reshape(B * P),
                          masko.reshape(B * P))
    y = _tail(out_core.reshape(PIX, C), Wout, r1(bout))
    return y.reshape(N, H, W, C)
